# baseline (speedup 1.0000x reference)
import functools

import jax
import jax.numpy as jnp
from jax import lax
from jax.experimental import pallas as pl
from jax.experimental.pallas import tpu as pltpu
from jax.experimental.pallas import tpu_sc as plsc


_PACK = 64
_ROWS_PER_BLOCK = 1024


def _score_body(table_ref, w_ref, b_ref, s_ref):
    blk = jnp.dot(
        table_ref[...], w_ref[...], preferred_element_type=jnp.float32
    ) + b_ref[...]
    b3 = blk.reshape(_ROWS_PER_BLOCK // 2, 2, _PACK)
    s_ref[...] = jnp.concatenate([b3[:, 0, :], b3[:, 1, :]], axis=1)


def _scores(table, W, b):
    n, h = table.shape
    t4 = table.reshape(n // _PACK, h * _PACK)
    wk = jnp.kron(jnp.eye(_PACK, dtype=W.dtype), W)
    grid = -(-(n // _PACK) // _ROWS_PER_BLOCK)
    out_rows = grid * _ROWS_PER_BLOCK // 2
    return pl.pallas_call(
        _score_body,
        grid=(grid,),
        in_specs=[
            pl.BlockSpec((_ROWS_PER_BLOCK, h * _PACK), lambda i: (i, 0)),
            pl.BlockSpec((h * _PACK, _PACK), lambda i: (0, 0)),
            pl.BlockSpec((1,), lambda i: (0,)),
        ],
        out_specs=pl.BlockSpec((_ROWS_PER_BLOCK // 2, 2 * _PACK), lambda i: (i, 0)),
        out_shape=jax.ShapeDtypeStruct((out_rows, 2 * _PACK), jnp.float32),
    )(t4, wk, b)


@functools.lru_cache(maxsize=None)
def _make_gather(bsz: int, hist: int):
    info = plsc.get_sparse_core_info()
    nc, ns, lanes = info.num_cores, info.num_subcores, info.num_lanes
    nw = nc * ns
    assert bsz % nw == 0
    rows_w = bsz // nw
    n_per_w = rows_w * hist
    assert n_per_w % lanes == 0

    mesh = plsc.VectorSubcoreMesh(core_axis_name="c", subcore_axis_name="s")
    fire = 16
    assert rows_w % fire == 0

    @functools.partial(
        pl.kernel,
        mesh=mesh,
        out_type=jax.ShapeDtypeStruct((bsz, hist), jnp.float32),
        scratch_types=[
            pltpu.VMEM((rows_w, hist), jnp.int32),
            pltpu.VMEM((rows_w, hist), jnp.float32),
            pltpu.SemaphoreType.DMA,
        ],
    )
    def gather_k(s_hbm, x_hbm, out_hbm, xv, val_v, sem):
        wid = lax.axis_index("s") * nc + lax.axis_index("c")
        base = wid * rows_w
        pltpu.sync_copy(x_hbm.at[pl.ds(base, rows_w)], xv)

        def batch(g, _):
            r0 = g * fire
            for i in range(fire):
                pltpu.async_copy(s_hbm.at[xv.at[r0 + i]], val_v.at[r0 + i], sem)
            for i in range(fire):
                pltpu.make_async_copy(
                    s_hbm.at[xv.at[r0 + i]], val_v.at[r0 + i], sem
                ).wait()
            return 0

        lax.fori_loop(0, rows_w // fire, batch, 0)
        pltpu.sync_copy(val_v, out_hbm.at[pl.ds(base, rows_w)])

    return gather_k


def kernel(x, table, W, b):
    bsz, hist = x.shape
    s = _scores(table, W, b).reshape(-1)
    y = _make_gather(bsz, hist)(s, x.astype(jnp.int32))
    return y.reshape(bsz, hist, 1)


# speedup vs baseline: 4.4834x; 4.4834x over previous
import functools

import jax
import jax.numpy as jnp
from jax import lax
from jax.experimental import pallas as pl
from jax.experimental.pallas import tpu as pltpu
from jax.experimental.pallas import tpu_sc as plsc


_ITEMS_PER_BLOCK = 65536


def _score_body(t_ref, w_ref, s_ref):
    s_ref[...] = jnp.sum(t_ref[...] * w_ref[...], axis=0)


def _scores(table, W):
    n, h = table.shape
    tt = jnp.swapaxes(table, 0, 1)
    grid = -(-n // _ITEMS_PER_BLOCK)
    return pl.pallas_call(
        _score_body,
        grid=(grid,),
        in_specs=[
            pl.BlockSpec((h, _ITEMS_PER_BLOCK), lambda i: (0, i)),
            pl.BlockSpec((h, 1), lambda i: (0, 0)),
        ],
        out_specs=pl.BlockSpec((_ITEMS_PER_BLOCK,), lambda i: (i,)),
        out_shape=jax.ShapeDtypeStruct((n,), jnp.float32),
    )(tt, W)


@functools.lru_cache(maxsize=None)
def _make_gather(bsz: int, hist: int):
    info = plsc.get_sparse_core_info()
    nc, ns, lanes = info.num_cores, info.num_subcores, info.num_lanes
    nw = nc * ns
    assert bsz % nw == 0
    rows_w = bsz // nw
    n_per_w = rows_w * hist
    assert n_per_w % lanes == 0

    mesh = plsc.VectorSubcoreMesh(core_axis_name="c", subcore_axis_name="s")

    @functools.partial(
        pl.kernel,
        mesh=mesh,
        out_type=jax.ShapeDtypeStruct((bsz * hist,), jnp.float32),
        scratch_types=[
            pltpu.VMEM((n_per_w,), jnp.int32),
            pltpu.VMEM((n_per_w,), jnp.float32),
            pltpu.SemaphoreType.DMA,
        ],
    )
    def gather_k(s_hbm, idx_hbm, out_hbm, idx_v, val_v, sem):
        wid = lax.axis_index("s") * nc + lax.axis_index("c")
        base = wid * n_per_w
        pltpu.sync_copy(idx_hbm.at[pl.ds(base, n_per_w)], idx_v)
        pltpu.async_copy(s_hbm.at[idx_v], val_v, sem).wait()
        pltpu.sync_copy(val_v, out_hbm.at[pl.ds(base, n_per_w)])

    return gather_k


def kernel(x, table, W, b):
    bsz, hist = x.shape
    s = _scores(table, W)
    idx = jnp.minimum(x.reshape(-1).astype(jnp.int32), table.shape[0] - 1)
    y1 = _make_gather(bsz, hist)(s, idx)
    return y1.reshape(bsz, hist, 1) + b[0]


# speedup vs baseline: 5.0868x; 1.1346x over previous
import functools

import jax
import jax.numpy as jnp
from jax import lax
from jax.experimental import pallas as pl
from jax.experimental.pallas import tpu as pltpu
from jax.experimental.pallas import tpu_sc as plsc


_ITEMS_PER_BLOCK = 65536


def _score_body(t_ref, w_ref, s_ref):
    s_ref[...] = jnp.sum(t_ref[...] * w_ref[...], axis=0)


def _scores(table, W):
    n, h = table.shape
    tt = jnp.swapaxes(table, 0, 1)
    grid = -(-n // _ITEMS_PER_BLOCK)
    return pl.pallas_call(
        _score_body,
        grid=(grid,),
        in_specs=[
            pl.BlockSpec((h, _ITEMS_PER_BLOCK), lambda i: (0, i)),
            pl.BlockSpec((h, 1), lambda i: (0, 0)),
        ],
        out_specs=pl.BlockSpec((_ITEMS_PER_BLOCK,), lambda i: (i,)),
        out_shape=jax.ShapeDtypeStruct((n,), jnp.float32),
    )(tt, W)


@functools.lru_cache(maxsize=None)
def _make_gather(bsz: int, hist: int):
    info = plsc.get_sparse_core_info()
    nc, ns, lanes = info.num_cores, info.num_subcores, info.num_lanes
    nw = nc * ns
    assert bsz % nw == 0
    rows_w = bsz // nw
    n_per_w = rows_w * hist
    assert n_per_w % lanes == 0

    mesh = plsc.VectorSubcoreMesh(core_axis_name="c", subcore_axis_name="s")

    @functools.partial(
        pl.kernel,
        mesh=mesh,
        out_type=jax.ShapeDtypeStruct((bsz * hist,), jnp.float32),
        scratch_types=[
            pltpu.VMEM((n_per_w,), jnp.int32),
            pltpu.VMEM((n_per_w,), jnp.float32),
            pltpu.SemaphoreType.DMA,
        ],
    )
    def gather_k(s_hbm, idx_hbm, out_hbm, idx_v, val_v, sem):
        wid = lax.axis_index("s") * nc + lax.axis_index("c")
        base = wid * n_per_w
        pltpu.sync_copy(idx_hbm.at[pl.ds(base, n_per_w)], idx_v)
        pltpu.async_copy(s_hbm.at[idx_v], val_v, sem).wait()
        pltpu.sync_copy(val_v, out_hbm.at[pl.ds(base, n_per_w)])

    return gather_k


def kernel(x, table, W, b):
    bsz, hist = x.shape
    s = _scores(table, W)
    idx_t = jnp.minimum(
        jnp.swapaxes(x, 0, 1).reshape(-1).astype(jnp.int32), table.shape[0] - 1
    )
    y1t = _make_gather(bsz, hist)(s, idx_t)
    return jnp.swapaxes(y1t.reshape(hist, bsz), 0, 1)[..., None] + b[0]
